# baseline (speedup 1.0000x reference)
import functools

import jax
import jax.numpy as jnp
from jax import lax
from jax.experimental import pallas as pl
from jax.experimental.pallas import tpu as pltpu
from jax.experimental.pallas import tpu_sc as plsc

V = 100000
NROWS = 512
R = 16
NRB = NROWS // R
WS = 128
NFULL = V // WS
TAIL_VALID = V - NFULL * WS
PAD_W = (NFULL + 1) * WS
NSTATES = 4
MIN = float(jnp.finfo(jnp.float32).min)
LOG2E = 1.4426950408889634


def _exp(z):
    return jnp.exp2(z * LOG2E)

SC_NC = 2
SC_NS = 16
SC_NW = SC_NC * SC_NS
SC_RPW = NROWS // SC_NW


def _insert5(state, x):
    r1, r2, r3, r4, r5 = state
    m1 = jnp.maximum(r1, x)
    v1 = jnp.minimum(r1, x)
    m2 = jnp.maximum(r2, v1)
    v2 = jnp.minimum(r2, v1)
    m3 = jnp.maximum(r3, v2)
    v3 = jnp.minimum(r3, v2)
    m4 = jnp.maximum(r4, v3)
    v4 = jnp.minimum(r4, v3)
    m5 = jnp.maximum(r5, v4)
    return (m1, m2, m3, m4, m5)


def _insert_pair(st, xa, xb):
    r1, r2, r3, r4, r5, q1, q2 = st
    pm = jnp.maximum(xa, xb)
    pn = jnp.minimum(xa, xb)
    r1, r2, r3, r4, r5 = _insert5((r1, r2, r3, r4, r5), pm)
    nq1 = jnp.maximum(q1, pn)
    qm = jnp.minimum(q1, pn)
    nq2 = jnp.maximum(q2, qm)
    return [r1, r2, r3, r4, r5, nq1, nq2]


def _make_tc_body(rows, ws, nfull, tail_valid, v, nrb, nstates=NSTATES,
                  npair=2):
    def body(x_ref, t_ref, out_ref, acc_ref):
        rb = pl.program_id(0)
        lane = lax.broadcasted_iota(jnp.int32, (rows, ws), 1)
        u = nstates
        nloop = nfull // u

        step = 2 * npair
        n1 = nfull // step

        def p1(j, c):
            sts = [list(c[7 * i:7 * i + 7]) for i in range(npair)]
            base = pl.multiple_of(j * (step * ws), ws)
            for i in range(npair):
                xa = x_ref[:, pl.ds(base + (2 * i) * ws, ws)]
                xb = x_ref[:, pl.ds(base + (2 * i + 1) * ws, ws)]
                sts[i] = _insert_pair(sts[i], xa, xb)
            return tuple(x for st in sts for x in st)

        init = tuple(jnp.full((rows, ws), MIN, jnp.float32)
                     for _ in range(7 * npair))
        st = lax.fori_loop(0, n1, p1, init)
        sts = [list(st[7 * i:7 * i + 7]) for i in range(npair)]
        rem = list(range(n1 * step, nfull))
        for i, jj in enumerate(rem):
            x = x_ref[:, pl.ds(jj * ws, ws)]
            sts[i % npair][:5] = _insert5(tuple(sts[i % npair][:5]), x)
        xt = x_ref[:, pl.ds(nfull * ws, ws)]
        xt = jnp.where(lane < tail_valid, xt, MIN)
        i = len(rem) % npair
        sts[i][:5] = _insert5(tuple(sts[i][:5]), xt)

        r1 = functools.reduce(jnp.maximum, [s[0] for s in sts])

        def p2(j, c):
            accs = list(c)
            base = pl.multiple_of(j * (u * ws), ws)
            for i in range(u):
                x = x_ref[:, pl.ds(base + i * ws, ws)]
                accs[i] = accs[i] + _exp(x - r1)
            return tuple(accs)

        zero = jnp.zeros((rows, ws), jnp.float32)
        accs = list(lax.fori_loop(0, nloop, p2, (zero,) * u))
        for i, jj in enumerate(range(nloop * u, nfull)):
            x = x_ref[:, pl.ds(jj * ws, ws)]
            accs[i % u] = accs[i % u] + _exp(x - r1)
        xt2 = x_ref[:, pl.ds(nfull * ws, ws)]
        accs[0] = accs[0] + jnp.where(lane < tail_valid, _exp(xt2 - r1), 0.0)
        s = functools.reduce(jnp.add, accs)

        lane1 = lax.broadcasted_iota(jnp.int32, (1, ws), 1)
        tvals = []
        for r in range(rows):
            tc = t_ref[rb * rows + r]
            sl = x_ref[pl.ds(r, 1), pl.ds(pl.multiple_of(tc & ~127, 128), ws)]
            tvals.append(jnp.sum(jnp.where(lane1 == (tc & 127), sl, 0.0),
                                 axis=1, keepdims=True))
        tpos = jnp.concatenate(tvals, axis=0)
        m_row = jnp.max(r1, axis=1, keepdims=True)
        s_row = jnp.sum(s * _exp(r1 - m_row), axis=1, keepdims=True)

        cur = [x for stt in sts for x in stt]
        vs = []
        for k in range(5):
            cm = functools.reduce(jnp.maximum, cur)
            vk = jnp.max(cm, axis=1, keepdims=True)
            vs.append(vk)
            if k < 4:
                cur = [jnp.where(a == vk, MIN, a) for a in cur]

        pos = _exp(tpos - m_row) / s_row
        sum5 = functools.reduce(
            jnp.add, [_exp(x - m_row) for x in vs]) / s_row
        in5 = jnp.where(tpos >= vs[4], 1.0, 0.0)
        cnt = 5.0 - in5
        sneg = sum5 - in5 * pos
        loss = -(cnt * pos - sneg) / cnt
        bsum = jnp.sum(loss)
        prev = jnp.where(rb == 0, 0.0, acc_ref[0])
        acc_ref[0] = prev + bsum

        @pl.when(rb == nrb - 1)
        def _():
            val = acc_ref[0] * jnp.float32(1.0 / (rows * nrb))
            out_ref[...] = jnp.full((1, 1), val, jnp.float32)

    return body


_tc_body = _make_tc_body(R, WS, NFULL, TAIL_VALID, V, NRB)


def _tc_loss(logits2, tgt):
    return pl.pallas_call(
        _tc_body,
        grid=(NRB,),
        in_specs=[
            pl.BlockSpec((R, PAD_W), lambda i: (i, 0)),
            pl.BlockSpec(memory_space=pltpu.SMEM),
        ],
        out_specs=pl.BlockSpec((1, 1), lambda i: (0, 0)),
        out_shape=jax.ShapeDtypeStruct((1, 1), jnp.float32),
        scratch_shapes=[pltpu.SMEM((1,), jnp.float32)],
    )(logits2, tgt)


def _sc_loss(tp, m, s, v1, v2, v3, v4, v5):
    mesh = plsc.VectorSubcoreMesh(core_axis_name="c", subcore_axis_name="s")

    @functools.partial(
        pl.kernel,
        mesh=mesh,
        out_type=jax.ShapeDtypeStruct((NROWS,), jnp.float32),
        scratch_types=[pltpu.VMEM((SC_RPW,), jnp.float32) for _ in range(9)],
    )
    def k(tp_h, m_h, s_h, v1_h, v2_h, v3_h, v4_h, v5_h, out,
          tp_v, m_v, s_v, v1_v, v2_v, v3_v, v4_v, v5_v, o_v):
        wid = lax.axis_index("s") * SC_NC + lax.axis_index("c")
        base = wid * SC_RPW
        ins = (tp_h, m_h, s_h, v1_h, v2_h, v3_h, v4_h, v5_h)
        scr = (tp_v, m_v, s_v, v1_v, v2_v, v3_v, v4_v, v5_v)
        for h, vv in zip(ins, scr):
            pltpu.sync_copy(h.at[pl.ds(base, SC_RPW)], vv)
        tpv, mv, sv = tp_v[...], m_v[...], s_v[...]
        vals = [v1_v[...], v2_v[...], v3_v[...], v4_v[...], v5_v[...]]
        pos = jnp.exp(tpv - mv) / sv
        sum5 = functools.reduce(
            jnp.add, [jnp.exp(x - mv) for x in vals]) / sv
        in5 = jnp.where(tpv >= vals[4], 1.0, 0.0)
        cnt = 5.0 - in5
        sneg = sum5 - in5 * pos
        o_v[...] = -(cnt * pos - sneg) / cnt
        pltpu.sync_copy(o_v, out.at[pl.ds(base, SC_RPW)])

    return k(tp, m, s, v1, v2, v3, v4, v5)


def _mean_body(x_ref, o_ref):
    o_ref[...] = jnp.full((1, 1), jnp.sum(x_ref[...]) *
                          jnp.float32(1.0 / NROWS), jnp.float32)


def _tc_mean(loss3):
    return pl.pallas_call(
        _mean_body,
        grid=(1,),
        in_specs=[pl.BlockSpec((4, 1, 128), lambda i: (0, 0, 0))],
        out_specs=pl.BlockSpec((1, 1), lambda i: (0, 0)),
        out_shape=jax.ShapeDtypeStruct((1, 1), jnp.float32),
    )(loss3)


def kernel(logits, target):
    b, sq, v = logits.shape
    logits2 = logits.reshape(b * sq, v)
    tgt = target.reshape(-1).astype(jnp.int32)
    res = _tc_loss(logits2, tgt)
    return res[0, 0]


# speedup vs baseline: 1.1987x; 1.1987x over previous
import functools

import jax
import jax.numpy as jnp
from jax import lax
from jax.experimental import pallas as pl
from jax.experimental.pallas import tpu as pltpu
from jax.experimental.pallas import tpu_sc as plsc

V = 100000
NROWS = 512
R = 16
NRB = NROWS // R
WS = 128
NFULL = V // WS
TAIL_VALID = V - NFULL * WS
PAD_W = (NFULL + 1) * WS
NSTATES = 4
MIN = float(jnp.finfo(jnp.float32).min)
LOG2E = 1.4426950408889634


def _exp(z):
    return jnp.exp2(z * LOG2E)

SC_NC = 2
SC_NS = 16
SC_NW = SC_NC * SC_NS
SC_RPW = NROWS // SC_NW


def _insert5(state, x):
    r1, r2, r3, r4, r5 = state
    m1 = jnp.maximum(r1, x)
    v1 = jnp.minimum(r1, x)
    m2 = jnp.maximum(r2, v1)
    v2 = jnp.minimum(r2, v1)
    m3 = jnp.maximum(r3, v2)
    v3 = jnp.minimum(r3, v2)
    m4 = jnp.maximum(r4, v3)
    v4 = jnp.minimum(r4, v3)
    m5 = jnp.maximum(r5, v4)
    return (m1, m2, m3, m4, m5)


def _insert_pair(st, xa, xb):
    r1, r2, r3, r4, r5, q1, q2 = st
    pm = jnp.maximum(xa, xb)
    pn = jnp.minimum(xa, xb)
    r1, r2, r3, r4, r5 = _insert5((r1, r2, r3, r4, r5), pm)
    nq1 = jnp.maximum(q1, pn)
    qm = jnp.minimum(q1, pn)
    nq2 = jnp.maximum(q2, qm)
    return [r1, r2, r3, r4, r5, nq1, nq2]


def _make_tc_body(rows, ws, nfull, tail_valid, v, nrb, nstates=NSTATES,
                  npair=3):
    def body(x_ref, t_ref, out_ref, acc_ref):
        rb = pl.program_id(0)
        lane = lax.broadcasted_iota(jnp.int32, (rows, ws), 1)
        u = nstates
        nloop = nfull // u

        step = 2 * npair
        n1 = nfull // step

        def p1(j, c):
            sts = [list(c[7 * i:7 * i + 7]) for i in range(npair)]
            base = pl.multiple_of(j * (step * ws), ws)
            for i in range(npair):
                xa = x_ref[:, pl.ds(base + (2 * i) * ws, ws)]
                xb = x_ref[:, pl.ds(base + (2 * i + 1) * ws, ws)]
                sts[i] = _insert_pair(sts[i], xa, xb)
            return tuple(x for st in sts for x in st)

        init = tuple(jnp.full((rows, ws), MIN, jnp.float32)
                     for _ in range(7 * npair))
        st = lax.fori_loop(0, n1, p1, init)
        sts = [list(st[7 * i:7 * i + 7]) for i in range(npair)]
        rem = list(range(n1 * step, nfull))
        for i, jj in enumerate(rem):
            x = x_ref[:, pl.ds(jj * ws, ws)]
            sts[i % npair][:5] = _insert5(tuple(sts[i % npair][:5]), x)
        xt = x_ref[:, pl.ds(nfull * ws, ws)]
        xt = jnp.where(lane < tail_valid, xt, MIN)
        i = len(rem) % npair
        sts[i][:5] = _insert5(tuple(sts[i][:5]), xt)

        r1 = functools.reduce(jnp.maximum, [s[0] for s in sts])

        cur = [x for stt in sts for x in stt]
        vs = []
        for k in range(5):
            cm = functools.reduce(jnp.maximum, cur)
            vk = jnp.max(cm, axis=1, keepdims=True)
            vs.append(vk)
            if k < 4:
                cur = [jnp.where(a == vk, MIN, a) for a in cur]

        ws2 = 2 * ws
        nfull2 = v // ws2
        lane2 = lax.broadcasted_iota(jnp.int32, (rows, ws2), 1)
        tail2 = v - nfull2 * ws2
        r1c = jnp.concatenate([r1, r1], axis=1)

        def p2(j, c):
            accs = list(c)
            base = pl.multiple_of(j * (u * ws2), ws2)
            for i in range(u):
                x = x_ref[:, pl.ds(base + i * ws2, ws2)]
                accs[i] = accs[i] + _exp(x - r1c)
            return tuple(accs)

        zero = jnp.zeros((rows, ws2), jnp.float32)
        accs = list(lax.fori_loop(0, nfull2 // u, p2, (zero,) * u))
        for i, jj in enumerate(range((nfull2 // u) * u, nfull2)):
            x = x_ref[:, pl.ds(jj * ws2, ws2)]
            accs[i % u] = accs[i % u] + _exp(x - r1c)
        xt2 = x_ref[:, pl.ds(nfull2 * ws2, ws2)]
        accs[0] = accs[0] + jnp.where(lane2 < tail2, _exp(xt2 - r1c), 0.0)
        s2 = functools.reduce(jnp.add, accs)
        s = s2[:, :ws] + s2[:, ws:]

        lane1 = lax.broadcasted_iota(jnp.int32, (1, ws), 1)
        tvals = []
        for r in range(rows):
            tc = t_ref[rb * rows + r]
            sl = x_ref[pl.ds(r, 1), pl.ds(pl.multiple_of(tc & ~127, 128), ws)]
            tvals.append(jnp.sum(jnp.where(lane1 == (tc & 127), sl, 0.0),
                                 axis=1, keepdims=True))
        tpos = jnp.concatenate(tvals, axis=0)
        m_row = jnp.max(r1, axis=1, keepdims=True)
        s_row = jnp.sum(s * _exp(r1 - m_row), axis=1, keepdims=True)

        pos = _exp(tpos - m_row) / s_row
        sum5 = functools.reduce(
            jnp.add, [_exp(x - m_row) for x in vs]) / s_row
        in5 = jnp.where(tpos >= vs[4], 1.0, 0.0)
        cnt = 5.0 - in5
        sneg = sum5 - in5 * pos
        loss = -(cnt * pos - sneg) / cnt
        bsum = jnp.sum(loss)
        prev = jnp.where(rb == 0, 0.0, acc_ref[0])
        acc_ref[0] = prev + bsum

        @pl.when(rb == nrb - 1)
        def _():
            val = acc_ref[0] * jnp.float32(1.0 / (rows * nrb))
            out_ref[...] = jnp.full((1, 1), val, jnp.float32)

    return body


_tc_body = _make_tc_body(R, WS, NFULL, TAIL_VALID, V, NRB)


def _tc_loss(logits2, tgt):
    return pl.pallas_call(
        _tc_body,
        grid=(NRB,),
        in_specs=[
            pl.BlockSpec((R, PAD_W), lambda i: (i, 0)),
            pl.BlockSpec(memory_space=pltpu.SMEM),
        ],
        out_specs=pl.BlockSpec((1, 1), lambda i: (0, 0)),
        out_shape=jax.ShapeDtypeStruct((1, 1), jnp.float32),
        scratch_shapes=[pltpu.SMEM((1,), jnp.float32)],
    )(logits2, tgt)


def _sc_loss(tp, m, s, v1, v2, v3, v4, v5):
    mesh = plsc.VectorSubcoreMesh(core_axis_name="c", subcore_axis_name="s")

    @functools.partial(
        pl.kernel,
        mesh=mesh,
        out_type=jax.ShapeDtypeStruct((NROWS,), jnp.float32),
        scratch_types=[pltpu.VMEM((SC_RPW,), jnp.float32) for _ in range(9)],
    )
    def k(tp_h, m_h, s_h, v1_h, v2_h, v3_h, v4_h, v5_h, out,
          tp_v, m_v, s_v, v1_v, v2_v, v3_v, v4_v, v5_v, o_v):
        wid = lax.axis_index("s") * SC_NC + lax.axis_index("c")
        base = wid * SC_RPW
        ins = (tp_h, m_h, s_h, v1_h, v2_h, v3_h, v4_h, v5_h)
        scr = (tp_v, m_v, s_v, v1_v, v2_v, v3_v, v4_v, v5_v)
        for h, vv in zip(ins, scr):
            pltpu.sync_copy(h.at[pl.ds(base, SC_RPW)], vv)
        tpv, mv, sv = tp_v[...], m_v[...], s_v[...]
        vals = [v1_v[...], v2_v[...], v3_v[...], v4_v[...], v5_v[...]]
        pos = jnp.exp(tpv - mv) / sv
        sum5 = functools.reduce(
            jnp.add, [jnp.exp(x - mv) for x in vals]) / sv
        in5 = jnp.where(tpv >= vals[4], 1.0, 0.0)
        cnt = 5.0 - in5
        sneg = sum5 - in5 * pos
        o_v[...] = -(cnt * pos - sneg) / cnt
        pltpu.sync_copy(o_v, out.at[pl.ds(base, SC_RPW)])

    return k(tp, m, s, v1, v2, v3, v4, v5)


def _mean_body(x_ref, o_ref):
    o_ref[...] = jnp.full((1, 1), jnp.sum(x_ref[...]) *
                          jnp.float32(1.0 / NROWS), jnp.float32)


def _tc_mean(loss3):
    return pl.pallas_call(
        _mean_body,
        grid=(1,),
        in_specs=[pl.BlockSpec((4, 1, 128), lambda i: (0, 0, 0))],
        out_specs=pl.BlockSpec((1, 1), lambda i: (0, 0)),
        out_shape=jax.ShapeDtypeStruct((1, 1), jnp.float32),
    )(loss3)


def kernel(logits, target):
    b, sq, v = logits.shape
    logits2 = logits.reshape(b * sq, v)
    tgt = target.reshape(-1).astype(jnp.int32)
    res = _tc_loss(logits2, tgt)
    return res[0, 0]
